# Initial kernel scaffold; baseline (speedup 1.0000x reference)
import jax
import jax.numpy as jnp
from jax.experimental import pallas as pl


def kernel(x, Wg, bg, W1, b1, W2, b2):
    raise NotImplementedError("write your pallas kernel here")


# speedup vs baseline: 4.2477x; 4.2477x over previous
import functools

import jax
import jax.numpy as jnp
from jax.experimental import pallas as pl
from jax.experimental.pallas import tpu as pltpu


def _moe_kernel(x_ref, wg_ref, bg_ref, w1_ref, b1_ref, w2_ref, b2_ref,
                out_ref, combine_ref, *, n_ff_chunks):
    e = pl.program_id(0)
    c = pl.program_id(1)

    @pl.when((e == 0) & (c == 0))
    def _gate():
        xb = x_ref[...]
        logits = jnp.dot(xb, wg_ref[...],
                         preferred_element_type=jnp.float32) + bg_ref[...]
        E = logits.shape[-1]
        col = jax.lax.broadcasted_iota(jnp.int32, logits.shape, 1)
        m1 = jnp.max(logits, axis=-1, keepdims=True)
        i1 = jnp.min(jnp.where(logits == m1, col, E), axis=-1, keepdims=True)
        rest = jnp.where(col == i1, -jnp.inf, logits)
        m2 = jnp.max(rest, axis=-1, keepdims=True)
        i2 = jnp.min(jnp.where(rest == m2, col, E), axis=-1, keepdims=True)
        e1 = jnp.exp(m1 - m1)
        e2 = jnp.exp(m2 - m1)
        s = e1 + e2
        w1 = e1 / s
        w2 = e2 / s
        combine_ref[...] = jnp.where(col == i1, w1, 0.0) + jnp.where(
            col == i2, w2, 0.0)
        out_ref[...] = jnp.zeros_like(out_ref)

    xb = x_ref[...]
    E = combine_ref.shape[-1]
    col = jax.lax.broadcasted_iota(jnp.int32, combine_ref.shape, 1)
    ce = jnp.sum(jnp.where(col == e, combine_ref[...], 0.0), axis=-1,
                 keepdims=True)

    h = jnp.dot(xb, w1_ref[0], preferred_element_type=jnp.float32) + b1_ref[0]
    h = 0.5 * h * (1.0 + jax.lax.erf(h * 0.7071067811865476))
    y = jnp.dot(h, w2_ref[0], preferred_element_type=jnp.float32)
    out_ref[...] += ce * y

    @pl.when(c == n_ff_chunks - 1)
    def _bias():
        out_ref[...] += ce * b2_ref[0]


def kernel(x, Wg, bg, W1, b1, W2, b2):
    B, S, H = x.shape
    T = B * S
    E, _, FF = W1.shape
    x_flat = x.reshape(T, H)

    ff_chunk = 512 if FF % 512 == 0 else FF
    n_ff = FF // ff_chunk

    out = pl.pallas_call(
        functools.partial(_moe_kernel, n_ff_chunks=n_ff),
        grid=(E, n_ff),
        in_specs=[
            pl.BlockSpec((T, H), lambda e, c: (0, 0)),
            pl.BlockSpec((H, E), lambda e, c: (0, 0)),
            pl.BlockSpec((1, E), lambda e, c: (0, 0)),
            pl.BlockSpec((1, H, ff_chunk), lambda e, c: (e, 0, c)),
            pl.BlockSpec((1, 1, ff_chunk), lambda e, c: (e, 0, c)),
            pl.BlockSpec((1, ff_chunk, H), lambda e, c: (e, c, 0)),
            pl.BlockSpec((1, 1, H), lambda e, c: (e, 0, 0)),
        ],
        out_specs=pl.BlockSpec((T, H), lambda e, c: (0, 0)),
        out_shape=jax.ShapeDtypeStruct((T, H), x.dtype),
        scratch_shapes=[pltpu.VMEM((T, E), jnp.float32)],
    )(x_flat, Wg, bg.reshape(1, E), W1, b1.reshape(E, 1, FF), W2,
      b2.reshape(E, 1, H))
    return out.reshape(B, S, H)


# speedup vs baseline: 5.6928x; 1.3402x over previous
import functools

import jax
import jax.numpy as jnp
from jax import lax
from jax.experimental import pallas as pl
from jax.experimental.pallas import tpu as pltpu
from jax.experimental.pallas import tpu_sc as plsc

TILE = 256
_NC = 2
_NS = 16
_NW = _NC * _NS


def _router_kernel(x_ref, wg_ref, bg_ref,
                   dest0_ref, dest1_ref, g0_ref, g1_ref,
                   te_ref, act_ref, src_ref, *, tile, max_tiles):
    T = x_ref.shape[0]
    E = wg_ref.shape[1]
    logits = jnp.dot(x_ref[...], wg_ref[...],
                     preferred_element_type=jnp.float32) + bg_ref[...]
    col = jax.lax.broadcasted_iota(jnp.int32, logits.shape, 1)
    m1 = jnp.max(logits, axis=-1, keepdims=True)
    i1 = jnp.min(jnp.where(logits == m1, col, E), axis=-1, keepdims=True)
    rest = jnp.where(col == i1, -jnp.inf, logits)
    m2 = jnp.max(rest, axis=-1, keepdims=True)
    i2 = jnp.min(jnp.where(rest == m2, col, E), axis=-1, keepdims=True)
    e2v = jnp.exp(m2 - m1)
    s = 1.0 + e2v
    g0_ref[...] = jnp.broadcast_to(1.0 / s, g0_ref.shape)
    g1_ref[...] = jnp.broadcast_to(e2v / s, g1_ref.shape)

    onehot1 = (col == i1).astype(jnp.float32)
    onehot2 = (col == i2).astype(jnp.float32)
    ind = onehot1 + onehot2

    r = jax.lax.broadcasted_iota(jnp.int32, (T, T), 0)
    c = jax.lax.broadcasted_iota(jnp.int32, (T, T), 1)
    L = (r > c).astype(jnp.float32)
    rank = jnp.dot(L, ind, preferred_element_type=jnp.float32)

    counts = jnp.sum(ind, axis=0, keepdims=True)
    ntiles = jnp.floor((counts + (tile - 1)) / tile)
    ec = jax.lax.broadcasted_iota(jnp.int32, (E, E), 0)
    er = jax.lax.broadcasted_iota(jnp.int32, (E, E), 1)
    U = (ec < er).astype(jnp.float32)
    tcum = jnp.dot(ntiles, U, preferred_element_type=jnp.float32)
    offsets = tile * tcum

    base = rank + offsets
    dest0_ref[...] = jnp.sum(base * onehot1, axis=-1,
                             keepdims=True).astype(jnp.int32)
    dest1_ref[...] = jnp.sum(base * onehot2, axis=-1,
                             keepdims=True).astype(jnp.int32)

    na = jnp.sum(ntiles)
    ti = jax.lax.broadcasted_iota(jnp.int32, (max_tiles, 1), 0).astype(
        jnp.float32)
    i_eff = jnp.minimum(ti, na - 1.0)
    tcum_b = jnp.broadcast_to(tcum, (max_tiles, E))
    te = jnp.sum((tcum_b <= i_eff).astype(jnp.float32), axis=-1,
                 keepdims=True) - 1.0
    te_ref[...] = te.astype(jnp.int32)
    act_ref[...] = (ti < na).astype(jnp.int32)
    src_ref[...] = i_eff.astype(jnp.int32)


def _dispatch_body(chunk, x_hbm, d0_hbm, d1_hbm, xs_hbm,
                   rows_v, i0_v, i1_v, sem0, sem1):
    wid = lax.axis_index("s") * _NC + lax.axis_index("c")
    base = wid * chunk
    pltpu.sync_copy(d0_hbm.at[pl.ds(base, chunk)], i0_v)
    pltpu.sync_copy(d1_hbm.at[pl.ds(base, chunk)], i1_v)
    pltpu.sync_copy(x_hbm.at[pl.ds(base, chunk)], rows_v)
    cp0 = pltpu.async_copy(rows_v, xs_hbm.at[i0_v], sem0)
    cp1 = pltpu.async_copy(rows_v, xs_hbm.at[i1_v], sem1)
    cp0.wait()
    cp1.wait()


def _ffn_kernel(te_ref, act_ref, src_ref,
                xs_ref, w1_ref, b1_ref, w2_ref, b2_ref, ys_ref):
    i = pl.program_id(0)

    @pl.when(act_ref[i] == 1)
    def _():
        h = jnp.dot(xs_ref[...], w1_ref[0],
                    preferred_element_type=jnp.float32) + b1_ref[0]
        h = 0.5 * h * (1.0 + jax.lax.erf(h * 0.7071067811865476))
        ys_ref[...] = jnp.dot(h, w2_ref[0],
                              preferred_element_type=jnp.float32) + b2_ref[0]


def _combine_body(chunk, H, ys_hbm, d0_hbm, d1_hbm, g0_hbm, g1_hbm, out_hbm,
                  rows0_v, rows1_v, i0_v, i1_v, g0_v, g1_v, sem0, sem1):
    wid = lax.axis_index("s") * _NC + lax.axis_index("c")
    base = wid * chunk
    pltpu.sync_copy(d0_hbm.at[pl.ds(base, chunk)], i0_v)
    pltpu.sync_copy(d1_hbm.at[pl.ds(base, chunk)], i1_v)
    pltpu.sync_copy(g0_hbm.at[pl.ds(base, chunk)], g0_v)
    pltpu.sync_copy(g1_hbm.at[pl.ds(base, chunk)], g1_v)
    cp0 = pltpu.async_copy(ys_hbm.at[i0_v], rows0_v, sem0)
    cp1 = pltpu.async_copy(ys_hbm.at[i1_v], rows1_v, sem1)
    cp0.wait()
    cp1.wait()

    def body(r, carry):
        a = g0_v[r]
        b = g1_v[r]
        for j in range(H // 16):
            sl = pl.ds(j * 16, 16)
            rows0_v[r, sl] = a * rows0_v[r, sl] + b * rows1_v[r, sl]
        return carry

    lax.fori_loop(0, chunk, body, 0)
    pltpu.sync_copy(rows0_v, out_hbm.at[pl.ds(base, chunk)])


def kernel(x, Wg, bg, W1, b1, W2, b2):
    B, S, H = x.shape
    T = B * S
    E, _, FF = W1.shape
    x_flat = x.reshape(T, H)
    max_tiles = 2 * T // TILE + E - 1
    NS_SLOTS = max_tiles * TILE
    chunk = T // _NW

    dest0, dest1, g0, g1, te, act, src = pl.pallas_call(
        functools.partial(_router_kernel, tile=TILE, max_tiles=max_tiles),
        in_specs=[
            pl.BlockSpec((T, H), lambda: (0, 0)),
            pl.BlockSpec((H, E), lambda: (0, 0)),
            pl.BlockSpec((1, E), lambda: (0, 0)),
        ],
        out_specs=[
            pl.BlockSpec((T, 1), lambda: (0, 0)),
            pl.BlockSpec((T, 1), lambda: (0, 0)),
            pl.BlockSpec((T, 16), lambda: (0, 0)),
            pl.BlockSpec((T, 16), lambda: (0, 0)),
            pl.BlockSpec((max_tiles, 1), lambda: (0, 0)),
            pl.BlockSpec((max_tiles, 1), lambda: (0, 0)),
            pl.BlockSpec((max_tiles, 1), lambda: (0, 0)),
        ],
        out_shape=[
            jax.ShapeDtypeStruct((T, 1), jnp.int32),
            jax.ShapeDtypeStruct((T, 1), jnp.int32),
            jax.ShapeDtypeStruct((T, 16), jnp.float32),
            jax.ShapeDtypeStruct((T, 16), jnp.float32),
            jax.ShapeDtypeStruct((max_tiles, 1), jnp.int32),
            jax.ShapeDtypeStruct((max_tiles, 1), jnp.int32),
            jax.ShapeDtypeStruct((max_tiles, 1), jnp.int32),
        ],
    )(x_flat, Wg, bg.reshape(1, E))

    d0 = dest0.reshape(T)
    d1 = dest1.reshape(T)

    mesh = plsc.VectorSubcoreMesh(core_axis_name="c", subcore_axis_name="s")

    xs = pl.kernel(
        functools.partial(_dispatch_body, chunk),
        out_type=jax.ShapeDtypeStruct((NS_SLOTS, H), jnp.float32),
        mesh=mesh,
        scratch_types=[
            pltpu.VMEM((chunk, H), jnp.float32),
            pltpu.VMEM((chunk,), jnp.int32),
            pltpu.VMEM((chunk,), jnp.int32),
            pltpu.SemaphoreType.DMA,
            pltpu.SemaphoreType.DMA,
        ],
    )(x_flat, d0, d1)

    ys = pl.pallas_call(
        _ffn_kernel,
        grid_spec=pltpu.PrefetchScalarGridSpec(
            num_scalar_prefetch=3,
            grid=(max_tiles,),
            in_specs=[
                pl.BlockSpec((TILE, H), lambda i, te, a, sr: (sr[i], 0)),
                pl.BlockSpec((1, H, FF), lambda i, te, a, sr: (te[i], 0, 0)),
                pl.BlockSpec((1, 1, FF), lambda i, te, a, sr: (te[i], 0, 0)),
                pl.BlockSpec((1, FF, H), lambda i, te, a, sr: (te[i], 0, 0)),
                pl.BlockSpec((1, 1, H), lambda i, te, a, sr: (te[i], 0, 0)),
            ],
            out_specs=pl.BlockSpec((TILE, H), lambda i, te, a, sr: (sr[i], 0)),
        ),
        out_shape=jax.ShapeDtypeStruct((NS_SLOTS, H), jnp.float32),
    )(te.reshape(-1), act.reshape(-1), src.reshape(-1),
      xs, W1, b1.reshape(E, 1, FF), W2, b2.reshape(E, 1, H))

    out = pl.kernel(
        functools.partial(_combine_body, chunk, H),
        out_type=jax.ShapeDtypeStruct((T, H), jnp.float32),
        mesh=mesh,
        scratch_types=[
            pltpu.VMEM((chunk, H), jnp.float32),
            pltpu.VMEM((chunk, H), jnp.float32),
            pltpu.VMEM((chunk,), jnp.int32),
            pltpu.VMEM((chunk,), jnp.int32),
            pltpu.VMEM((chunk, 16), jnp.float32),
            pltpu.VMEM((chunk, 16), jnp.float32),
            pltpu.SemaphoreType.DMA,
            pltpu.SemaphoreType.DMA,
        ],
    )(ys, d0, d1, g0, g1)

    return out.reshape(B, S, H)
